# baseline (speedup 1.0000x reference)
import functools

import jax
import jax.numpy as jnp
from jax import lax
from jax.experimental import pallas as pl
from jax.experimental.pallas import tpu as pltpu
from jax.experimental.pallas import tpu_sc as plsc

LJ_SIGMA = 0.01
LJ_EPSILON = 1.0
N_NODES = 50000
N_EDGES = 1600000

NC, NS, L = 2, 16, 16
NW = NC * NS
NPAD = 50176
SLICE = NPAD // NS
CHUNK = 2000
N_CHUNKS = 25
E_PER_W = CHUNK * N_CHUNKS

_mesh = plsc.VectorSubcoreMesh(core_axis_name="c", subcore_axis_name="s")


@functools.partial(
    pl.kernel,
    out_type=jax.ShapeDtypeStruct((NC * NPAD,), jnp.float32),
    mesh=_mesh,
    compiler_params=pltpu.CompilerParams(
        needs_layout_passes=False, use_tc_tiling_on_sc=False),
    scratch_types=[
        pltpu.VMEM((1, CHUNK), jnp.int32),
        pltpu.VMEM((1, CHUNK), jnp.int32),
        pltpu.VMEM((1, CHUNK), jnp.int32),
        pltpu.VMEM((1, CHUNK), jnp.int32),
        pltpu.VMEM((CHUNK, 3), jnp.float32),
        pltpu.VMEM((CHUNK, 3), jnp.float32),
        pltpu.VMEM((CHUNK, 3), jnp.float32),
        pltpu.VMEM((CHUNK, 3), jnp.float32),
        pltpu.VMEM((CHUNK,), jnp.float32),
        pltpu.VMEM((CHUNK,), jnp.float32),
        pltpu.VMEM((SLICE,), jnp.float32),
        pltpu.VMEM_SHARED((NPAD,), jnp.float32),
        pltpu.VMEM_SHARED((N_NODES, 3), jnp.float32),
        pltpu.SemaphoreType.DMA,
        pltpu.SemaphoreType.DMA,
        pltpu.SemaphoreType.DMA,
        pltpu.SemaphoreType.DMA,
        pltpu.SemaphoreType.DMA,
        pltpu.SemaphoreType.DMA,
    ],
)
def _lj_sc(pos3, eidx, out, si0, si1, di0, di1, sr0, sr1, dr0, dr1,
           en0, en1, stage_v, acc_sh, pos_sh, smi0, smi1, smg0, smg1,
           sms0, sms1):
    c = lax.axis_index("c")
    s = lax.axis_index("s")
    wid = c * NS + s

    si_v = (si0, si1)
    di_v = (di0, di1)
    sr_v = (sr0, sr1)
    dr_v = (dr0, dr1)
    en_v = (en0, en1)
    smi = (smi0, smi1)
    smg = (smg0, smg1)
    sms = (sms0, sms1)

    zero16 = jnp.zeros((L,), jnp.float32)

    def _zero(i, carry):
        stage_v[pl.ds(i * L, L)] = zero16
        return carry

    lax.fori_loop(0, SLICE // L, _zero, 0)
    pltpu.sync_copy(stage_v, acc_sh.at[pl.ds(s * SLICE, SLICE)])

    @pl.when(s == 0)
    def _():
        pltpu.sync_copy(pos3, pos_sh)

    plsc.subcore_barrier()

    iota = lax.iota(jnp.int32, L)
    col0 = jnp.zeros((L,), jnp.int32)
    col1 = jnp.full((L,), 1, jnp.int32)
    col2 = jnp.full((L,), 2, jnp.int32)
    sig2 = jnp.full((L,), LJ_SIGMA * LJ_SIGMA, jnp.float32)
    two_eps = jnp.full((L,), 2.0 * LJ_EPSILON, jnp.float32)

    base_e = wid * E_PER_W

    def idx_copies(ci, b):
        off = base_e + ci * CHUNK
        return [
            pltpu.make_async_copy(
                eidx.at[pl.ds(off, CHUNK)], si_v[b].at[0], smi[b]),
            pltpu.make_async_copy(
                eidx.at[pl.ds(N_EDGES + off, CHUNK)], di_v[b].at[0],
                smi[b]),
        ]

    def gather_copies(b):
        h = CHUNK // 2
        return [
            pltpu.make_async_copy(pos_sh.at[si_v[b].at[0, pl.ds(0, h)]],
                                  sr_v[b].at[pl.ds(0, h)], smg[b]),
            pltpu.make_async_copy(pos_sh.at[di_v[b].at[0, pl.ds(0, h)]],
                                  dr_v[b].at[pl.ds(0, h)], smg[b]),
            pltpu.make_async_copy(pos_sh.at[si_v[b].at[0, pl.ds(h, h)]],
                                  sr_v[b].at[pl.ds(h, h)], smg[b]),
            pltpu.make_async_copy(pos_sh.at[di_v[b].at[0, pl.ds(h, h)]],
                                  dr_v[b].at[pl.ds(h, h)], smg[b]),
        ]

    def start_scatter(b):
        pltpu.async_copy(en_v[b], acc_sh.at[si_v[b].at[0]], sms[b], add=True)

    def wait_scatter(b):
        pltpu.make_async_copy(en_v[b], acc_sh.at[si_v[b].at[0]],
                              sms[b]).wait()

    def prefetch(ci, b):
        icps = idx_copies(ci, b)
        for cp in icps:
            cp.start()
        for cp in icps:
            cp.wait()
        for cp in gather_copies(b):
            cp.start()

    def compute(b):
        @plsc.parallel_loop(0, CHUNK // L, unroll=4)
        def _group(g):
            rid = g * L + iota
            xs = plsc.load_gather(sr_v[b], [rid, col0])
            ys = plsc.load_gather(sr_v[b], [rid, col1])
            zs = plsc.load_gather(sr_v[b], [rid, col2])
            xd = plsc.load_gather(dr_v[b], [rid, col0])
            yd = plsc.load_gather(dr_v[b], [rid, col1])
            zd = plsc.load_gather(dr_v[b], [rid, col2])
            dx = xd - xs
            dy = yd - ys
            dz = zd - zs
            r2 = dx * dx + dy * dy + dz * dz
            t = sig2 / r2
            t3 = t * t * t
            eng = two_eps * (t3 * t3 - t3)
            en_v[b][pl.ds(g * L, L)] = eng

    prefetch(0, 0)

    def _step(st, carry):
        for b in (0, 1):
            ci = st * 2 + b

            @pl.when(ci >= 1)
            def _():
                wait_scatter(1 - b)

            prefetch(ci + 1, 1 - b)
            for cp in gather_copies(b):
                cp.wait()
            compute(b)
            start_scatter(b)
        return carry

    lax.fori_loop(0, (N_CHUNKS - 1) // 2, _step, 0)

    wait_scatter(1)
    for cp in gather_copies(0):
        cp.wait()
    compute(0)
    start_scatter(0)
    wait_scatter(0)

    plsc.subcore_barrier()
    pltpu.sync_copy(acc_sh.at[pl.ds(s * SLICE, SLICE)], stage_v)
    pltpu.sync_copy(stage_v, out.at[pl.ds(c * NPAD + s * SLICE, SLICE)])


def kernel(pos, edge_index):
    eidx_flat = edge_index.reshape(-1)
    partial = _lj_sc(pos, eidx_flat)
    return (partial[:N_NODES] + partial[NPAD:NPAD + N_NODES]).reshape(
        N_NODES, 1)


# speedup vs baseline: 1.2654x; 1.2654x over previous
import functools

import jax
import jax.numpy as jnp
from jax import lax
from jax.experimental import pallas as pl
from jax.experimental.pallas import tpu as pltpu
from jax.experimental.pallas import tpu_sc as plsc

LJ_SIGMA = 0.01
LJ_EPSILON = 1.0
N_NODES = 50000
N_EDGES = 1600000

NC, NS, L = 2, 16, 16
NW = NC * NS
NPAD = 50176
SLICE = NPAD // NS
CHUNK = 2000
N_CHUNKS = 25
E_PER_W = CHUNK * N_CHUNKS

_mesh = plsc.VectorSubcoreMesh(core_axis_name="c", subcore_axis_name="s")


@functools.partial(
    pl.kernel,
    out_type=jax.ShapeDtypeStruct((NC * NPAD,), jnp.float32),
    mesh=_mesh,
    compiler_params=pltpu.CompilerParams(
        needs_layout_passes=False, use_tc_tiling_on_sc=False),
    scratch_types=[
        pltpu.VMEM((1, CHUNK), jnp.int32),
        pltpu.VMEM((1, CHUNK), jnp.int32),
        pltpu.VMEM((1, CHUNK), jnp.int32),
        pltpu.VMEM((1, CHUNK), jnp.int32),
        pltpu.VMEM((CHUNK,), jnp.float32),
        pltpu.VMEM((CHUNK,), jnp.float32),
        pltpu.VMEM((CHUNK,), jnp.float32),
        pltpu.VMEM((CHUNK,), jnp.float32),
        pltpu.VMEM((N_NODES,), jnp.float32),
        pltpu.VMEM((N_NODES,), jnp.float32),
        pltpu.VMEM((CHUNK,), jnp.float32),
        pltpu.VMEM((CHUNK,), jnp.float32),
        pltpu.VMEM((SLICE,), jnp.float32),
        pltpu.VMEM_SHARED((NPAD,), jnp.float32),
        pltpu.VMEM_SHARED((N_NODES,), jnp.float32),
        pltpu.SemaphoreType.DMA,
        pltpu.SemaphoreType.DMA,
        pltpu.SemaphoreType.DMA,
        pltpu.SemaphoreType.DMA,
        pltpu.SemaphoreType.DMA,
        pltpu.SemaphoreType.DMA,
    ],
)
def _lj_sc(posT, eidx, out, si0, si1, di0, di1, sr0, sr1, dr0, dr1,
           xtab, ytab, en0, en1, stage_v, acc_sh, z_sh, smi0, smi1,
           smg0, smg1, sms0, sms1):
    c = lax.axis_index("c")
    s = lax.axis_index("s")
    wid = c * NS + s

    si_v = (si0, si1)
    di_v = (di0, di1)
    sr_v = (sr0, sr1)
    dr_v = (dr0, dr1)
    en_v = (en0, en1)
    smi = (smi0, smi1)
    smg = (smg0, smg1)
    sms = (sms0, sms1)

    zero16 = jnp.zeros((L,), jnp.float32)

    def _zero(i, carry):
        stage_v[pl.ds(i * L, L)] = zero16
        return carry

    lax.fori_loop(0, SLICE // L, _zero, 0)
    pltpu.sync_copy(stage_v, acc_sh.at[pl.ds(s * SLICE, SLICE)])

    pltpu.sync_copy(posT.at[0], xtab)
    pltpu.sync_copy(posT.at[1], ytab)

    @pl.when(s == 0)
    def _():
        pltpu.sync_copy(posT.at[2], z_sh)

    plsc.subcore_barrier()

    iota = lax.iota(jnp.int32, L)
    col0 = jnp.zeros((L,), jnp.int32)
    col1 = jnp.full((L,), 1, jnp.int32)
    col2 = jnp.full((L,), 2, jnp.int32)
    sig2 = jnp.full((L,), LJ_SIGMA * LJ_SIGMA, jnp.float32)
    two_eps = jnp.full((L,), 2.0 * LJ_EPSILON, jnp.float32)

    base_e = wid * E_PER_W

    def idx_copies(ci, b):
        off = base_e + ci * CHUNK
        return [
            pltpu.make_async_copy(
                eidx.at[pl.ds(off, CHUNK)], si_v[b].at[0], smi[b]),
            pltpu.make_async_copy(
                eidx.at[pl.ds(N_EDGES + off, CHUNK)], di_v[b].at[0],
                smi[b]),
        ]

    def gather_copies(b):
        return [
            pltpu.make_async_copy(z_sh.at[si_v[b].at[0]], sr_v[b], smg[b]),
            pltpu.make_async_copy(z_sh.at[di_v[b].at[0]], dr_v[b], smg[b]),
        ]

    def start_scatter(b):
        pltpu.async_copy(en_v[b], acc_sh.at[si_v[b].at[0]], sms[b], add=True)

    def wait_scatter(b):
        pltpu.make_async_copy(en_v[b], acc_sh.at[si_v[b].at[0]],
                              sms[b]).wait()

    def prefetch(ci, b):
        icps = idx_copies(ci, b)
        for cp in icps:
            cp.start()
        for cp in icps:
            cp.wait()
        for cp in gather_copies(b):
            cp.start()

    def compute(b):
        @plsc.parallel_loop(0, CHUNK // L, unroll=4)
        def _group(g):
            si16 = si_v[b][0, pl.ds(g * L, L)]
            di16 = di_v[b][0, pl.ds(g * L, L)]
            xs = plsc.load_gather(xtab, [si16])
            ys = plsc.load_gather(ytab, [si16])
            zs = sr_v[b][pl.ds(g * L, L)]
            xd = plsc.load_gather(xtab, [di16])
            yd = plsc.load_gather(ytab, [di16])
            zd = dr_v[b][pl.ds(g * L, L)]
            dx = xd - xs
            dy = yd - ys
            dz = zd - zs
            r2 = dx * dx + dy * dy + dz * dz
            t = sig2 / r2
            t3 = t * t * t
            eng = two_eps * (t3 * t3 - t3)
            en_v[b][pl.ds(g * L, L)] = eng

    prefetch(0, 0)

    def _step(st, carry):
        for b in (0, 1):
            ci = st * 2 + b

            @pl.when(ci >= 1)
            def _():
                wait_scatter(1 - b)

            prefetch(ci + 1, 1 - b)
            for cp in gather_copies(b):
                cp.wait()
            compute(b)
            start_scatter(b)
        return carry

    lax.fori_loop(0, (N_CHUNKS - 1) // 2, _step, 0)

    wait_scatter(1)
    for cp in gather_copies(0):
        cp.wait()
    compute(0)
    start_scatter(0)
    wait_scatter(0)

    plsc.subcore_barrier()
    pltpu.sync_copy(acc_sh.at[pl.ds(s * SLICE, SLICE)], stage_v)
    pltpu.sync_copy(stage_v, out.at[pl.ds(c * NPAD + s * SLICE, SLICE)])


def kernel(pos, edge_index):
    eidx_flat = edge_index.reshape(-1)
    partial = _lj_sc(pos.T, eidx_flat)
    return (partial[:N_NODES] + partial[NPAD:NPAD + N_NODES]).reshape(
        N_NODES, 1)
